# baseline (speedup 1.0000x reference)
import jax
import jax.numpy as jnp
from jax import lax
from jax.experimental import pallas as pl
from jax.experimental.pallas import tpu as pltpu
from jax.experimental.pallas import tpu_sc as plsc

BATCH = 4096
SEQ = 50
DIM = 64
PADDING_IDX = 0

NUM_CORES = 2
NUM_SUBCORES = 16
NUM_WORKERS = NUM_CORES * NUM_SUBCORES

BATCH_PER_WORKER = BATCH // NUM_WORKERS
CB = 2
CHUNKS_PER_WORKER = BATCH_PER_WORKER // CB
LANES = 16
COLV = DIM // LANES
NGROUPS = 4

NBUF = 8
ROUNDS = CHUNKS_PER_WORKER // NBUF


def _lookup_body(table_hbm, idx_hbm, out_hbm, idx_v, rows_v, *sems):
    gsems, ssems = sems[:NBUF], sems[NBUF:]
    wid = lax.axis_index("s") * NUM_CORES + lax.axis_index("c")
    b0 = wid * BATCH_PER_WORKER
    pltpu.sync_copy(idx_hbm.at[pl.ds(b0, BATCH_PER_WORKER)], idx_v)

    def gather_desc(j, slot, q):
        return pltpu.make_async_copy(
            table_hbm.at[idx_v.at[CB * j + q]],
            rows_v.at[slot].at[q], gsems[slot])

    def store_desc(j, slot):
        return pltpu.make_async_copy(
            rows_v.at[slot], out_hbm.at[pl.ds(b0 + CB * j, CB)], ssems[slot])

    def fixup(j, slot, q):
        bb = CB * j + q
        vmin = idx_v[bb, pl.ds(0, LANES)]
        vmin = jnp.minimum(vmin, idx_v[bb, pl.ds(16, LANES)])
        vmin = jnp.minimum(vmin, idx_v[bb, pl.ds(32, LANES)])
        vmin = jnp.minimum(vmin, idx_v[bb, pl.ds(SEQ - LANES, LANES)])
        smin = vmin[0]
        for i in range(1, LANES):
            smin = jnp.minimum(smin, vmin[i])

        @pl.when(smin == PADDING_IDX)
        def _fix():
            def grp_body(g, c2):
                off = jnp.minimum(g * LANES, SEQ - LANES)
                idx16 = idx_v[bb, pl.ds(off, LANES)]
                for i in range(LANES):

                    @pl.when(idx16[i] == PADDING_IDX)
                    def _zero(i=i):
                        r = off + i
                        for c in range(COLV):
                            rows_v[slot, q, r, pl.ds(c * LANES, LANES)] = (
                                jnp.zeros((LANES,), jnp.float32))

                return c2

            lax.fori_loop(0, NGROUPS, grp_body, 0)

    for b in range(NBUF):
        for q in range(CB):
            gather_desc(b, b, q).start()

    def round_body(t, carry):
        for b in range(NBUF):
            j = t * NBUF + b
            for q in range(CB):
                gather_desc(j, b, q).wait()
            for q in range(CB):
                fixup(j, b, q)
            store_desc(j, b).start()

        @pl.when(t < ROUNDS - 1)
        def _issue_next():
            for b in range(NBUF):
                j = t * NBUF + b
                store_desc(j, b).wait()
                for q in range(CB):
                    gather_desc(j + NBUF, b, q).start()

        return carry

    lax.fori_loop(0, ROUNDS, round_body, 0)

    for b in range(NBUF):
        store_desc((ROUNDS - 1) * NBUF + b, b).wait()


_lookup = pl.kernel(
    _lookup_body,
    out_type=jax.ShapeDtypeStruct((BATCH, SEQ, DIM), jnp.float32),
    mesh=plsc.VectorSubcoreMesh(core_axis_name="c", subcore_axis_name="s"),
    compiler_params=pltpu.CompilerParams(use_tc_tiling_on_sc=False),
    scratch_types=[
        pltpu.VMEM((BATCH_PER_WORKER, SEQ), jnp.int32),
        pltpu.VMEM((NBUF, CB, SEQ, DIM), jnp.float32),
    ] + [pltpu.SemaphoreType.DMA] * (2 * NBUF),
)


def kernel(input_batch, table):
    return _lookup(table, input_batch.astype(jnp.int32))


# speedup vs baseline: 1.1225x; 1.1225x over previous
import jax
import jax.numpy as jnp
from jax import lax
from jax.experimental import pallas as pl
from jax.experimental.pallas import tpu as pltpu
from jax.experimental.pallas import tpu_sc as plsc

BATCH = 4096
SEQ = 50
DIM = 64
PDIM = 128
PADDING_IDX = 0

NUM_CORES = 2
NUM_SUBCORES = 16
NUM_WORKERS = NUM_CORES * NUM_SUBCORES

BATCH_PER_WORKER = BATCH // NUM_WORKERS
LANES = 16
COLV = DIM // LANES
NGROUPS = 4

NBUF = 4
ROUNDS = BATCH_PER_WORKER // NBUF


def _lookup_body(table_hbm, idx_hbm, out_hbm, idx_v, g_v, s_v, *sems):
    gsems, ssems = sems[:NBUF], sems[NBUF:]
    wid = lax.axis_index("s") * NUM_CORES + lax.axis_index("c")
    b0 = wid * BATCH_PER_WORKER
    pltpu.sync_copy(idx_hbm.at[pl.ds(b0, BATCH_PER_WORKER)], idx_v)

    def gather_desc(bb, slot):
        return pltpu.make_async_copy(
            table_hbm.at[idx_v.at[bb]], g_v.at[slot], gsems[slot])

    def store_desc(bb, slot):
        return pltpu.make_async_copy(
            s_v.at[slot], out_hbm.at[b0 + bb], ssems[slot])

    def compact(slot):
        def pair_body(k, c1):
            r = 2 * k
            for rr in (0, 1):
                for c in range(COLV):
                    s_v[slot, r + rr, pl.ds(c * LANES, LANES)] = (
                        g_v[slot, r + rr, pl.ds(c * LANES, LANES)])
            return c1

        lax.fori_loop(0, SEQ // 2, pair_body, 0)

    def fixup(bb, slot):
        vmin = idx_v[bb, pl.ds(0, LANES)]
        vmin = jnp.minimum(vmin, idx_v[bb, pl.ds(16, LANES)])
        vmin = jnp.minimum(vmin, idx_v[bb, pl.ds(32, LANES)])
        vmin = jnp.minimum(vmin, idx_v[bb, pl.ds(SEQ - LANES, LANES)])
        smin = vmin[0]
        for i in range(1, LANES):
            smin = jnp.minimum(smin, vmin[i])

        @pl.when(smin == PADDING_IDX)
        def _fix():
            def grp_body(g, c2):
                off = jnp.minimum(g * LANES, SEQ - LANES)
                idx16 = idx_v[bb, pl.ds(off, LANES)]
                for i in range(LANES):

                    @pl.when(idx16[i] == PADDING_IDX)
                    def _zero(i=i):
                        for c in range(COLV):
                            s_v[slot, off + i, pl.ds(c * LANES, LANES)] = (
                                jnp.zeros((LANES,), jnp.float32))

                return c2

            lax.fori_loop(0, NGROUPS, grp_body, 0)

    for b in range(NBUF):
        gather_desc(b, b).start()

    def round_body(t, carry):
        for b in range(NBUF):
            bb = t * NBUF + b
            gather_desc(bb, b).wait()
            compact(b)
            fixup(bb, b)
            store_desc(bb, b).start()

        @pl.when(t < ROUNDS - 1)
        def _issue_next():
            for b in range(NBUF):
                bb = t * NBUF + b
                store_desc(bb, b).wait()
                gather_desc(bb + NBUF, b).start()

        return carry

    lax.fori_loop(0, ROUNDS, round_body, 0)

    for b in range(NBUF):
        store_desc((ROUNDS - 1) * NBUF + b, b).wait()


_lookup = pl.kernel(
    _lookup_body,
    out_type=jax.ShapeDtypeStruct((BATCH, SEQ, DIM), jnp.float32),
    mesh=plsc.VectorSubcoreMesh(core_axis_name="c", subcore_axis_name="s"),
    scratch_types=[
        pltpu.VMEM((BATCH_PER_WORKER, SEQ), jnp.int32),
        pltpu.VMEM((NBUF, SEQ, PDIM), jnp.float32),
        pltpu.VMEM((NBUF, SEQ, DIM), jnp.float32),
    ] + [pltpu.SemaphoreType.DMA] * (2 * NBUF),
)


def kernel(input_batch, table):
    padded = jnp.pad(table, ((0, 0), (0, PDIM - DIM)))
    return _lookup(padded, input_batch.astype(jnp.int32))
